# Initial kernel scaffold; baseline (speedup 1.0000x reference)
import jax
import jax.numpy as jnp
from jax.experimental import pallas as pl


def kernel(x, edge_index, edge_weight, W1, b1, W2, b2):
    raise NotImplementedError("write your pallas kernel here")


# speedup vs baseline: 16.8162x; 16.8162x over previous
import functools

import jax
import jax.numpy as jnp
from jax import lax
from jax.experimental import pallas as pl
from jax.experimental.pallas import tpu as pltpu
from jax.experimental.pallas import tpu_sc as plsc

_N = 10000
_E = 320000
_FIN = 128
_HID = 64
_CLS = 40
_CP = 48

_NCORE, _NSUB, _LANES = 2, 16, 16
_NW = _NCORE * _NSUB
_EPT = _E // _NW
_CHUNK = 80
_NCHUNK = _EPT // _CHUNK
_NGRP = _CHUNK // _LANES
_NPAD = 10240
_RPT = _NPAD // _NSUB
_ZROWS = 128

_MESH = dict(core_axis_name="c", subcore_axis_name="s")
_SC_PARAMS = pltpu.CompilerParams(
    use_tc_tiling_on_sc=False, needs_layout_passes=False
)


def _sc_deg(dst2, w2):
    @functools.partial(
        pl.kernel,
        mesh=plsc.VectorSubcoreMesh(**_MESH),
        out_type=jax.ShapeDtypeStruct((_NCORE, _NPAD, _LANES), jnp.float32),
        scratch_types=[
            pltpu.VMEM((_NCHUNK, _CHUNK), jnp.int32),
            pltpu.VMEM((_NCHUNK, _CHUNK), jnp.float32),
            pltpu.VMEM((_CHUNK, _LANES), jnp.float32),
            pltpu.VMEM((_ZROWS, _LANES), jnp.float32),
            pltpu.VMEM_SHARED((_NPAD, _LANES), jnp.float32),
        ],
        compiler_params=_SC_PARAMS,
    )
    def k(dst_hbm, w_hbm, out_hbm, dst_v, w_v, vbuf, zbuf, acc):
        c = lax.axis_index("c")
        s = lax.axis_index("s")
        wid = c * _NSUB + s
        zero = jnp.zeros((_LANES,), jnp.float32)

        def zrow(i, carry):
            zbuf[i, :] = zero
            return carry

        lax.fori_loop(0, _ZROWS, zrow, 0)
        row0 = s * _RPT
        for r in range(_RPT // _ZROWS):
            pltpu.sync_copy(zbuf, acc.at[pl.ds(row0 + r * _ZROWS, _ZROWS)])
        pltpu.sync_copy(dst_hbm.at[wid], dst_v)
        pltpu.sync_copy(w_hbm.at[wid], w_v)
        plsc.subcore_barrier()

        def chunk(ci, carry):
            def fill(g, carry2):
                wv = w_v[ci, pl.ds(g * _LANES, _LANES)]
                for kk in range(_LANES):
                    vbuf[g * _LANES + kk, :] = zero + wv[kk]
                return carry2

            lax.fori_loop(0, _NGRP, fill, 0)
            pltpu.sync_copy(vbuf, acc.at[dst_v.at[ci]], add=True)
            return carry

        lax.fori_loop(0, _NCHUNK, chunk, 0)
        plsc.subcore_barrier()
        for r in range(_RPT // _ZROWS):
            ro = row0 + r * _ZROWS
            pltpu.sync_copy(acc.at[pl.ds(ro, _ZROWS)], out_hbm.at[c, pl.ds(ro, _ZROWS)])

    return k(dst2, w2)


def _sc_agg(hs, src2, dst2, w2, d):
    nj = d // _LANES

    @functools.partial(
        pl.kernel,
        mesh=plsc.VectorSubcoreMesh(**_MESH),
        out_type=jax.ShapeDtypeStruct((_NCORE, _NPAD, d), jnp.float32),
        scratch_types=[
            pltpu.VMEM((_NCHUNK, _CHUNK), jnp.int32),
            pltpu.VMEM((_NCHUNK, _CHUNK), jnp.int32),
            pltpu.VMEM((_NCHUNK, _CHUNK), jnp.float32),
            pltpu.VMEM((_CHUNK, d), jnp.float32),
            pltpu.VMEM((_ZROWS, d), jnp.float32),
            pltpu.VMEM_SHARED((_NPAD, d), jnp.float32),
            pltpu.SemaphoreType.DMA,
        ],
        compiler_params=_SC_PARAMS,
    )
    def k(hs_hbm, src_hbm, dst_hbm, w_hbm, out_hbm,
          src_v, dst_v, w_v, rows_v, zbuf, acc, gsem):
        c = lax.axis_index("c")
        s = lax.axis_index("s")
        wid = c * _NSUB + s
        zero = jnp.zeros((_LANES,), jnp.float32)

        def zrow(i, carry):
            for j in range(nj):
                zbuf[i, pl.ds(j * _LANES, _LANES)] = zero
            return carry

        lax.fori_loop(0, _ZROWS, zrow, 0)
        row0 = s * _RPT
        for r in range(_RPT // _ZROWS):
            pltpu.sync_copy(zbuf, acc.at[pl.ds(row0 + r * _ZROWS, _ZROWS)])
        pltpu.sync_copy(src_hbm.at[wid], src_v)
        pltpu.sync_copy(dst_hbm.at[wid], dst_v)
        pltpu.sync_copy(w_hbm.at[wid], w_v)
        plsc.subcore_barrier()

        def chunk(ci, carry):
            pltpu.async_copy(hs_hbm.at[src_v.at[ci]], rows_v, gsem).wait()

            def mul(g, carry2):
                wv = w_v[ci, pl.ds(g * _LANES, _LANES)]
                for kk in range(_LANES):
                    wk = wv[kk]
                    e = g * _LANES + kk
                    for j in range(nj):
                        sl = pl.ds(j * _LANES, _LANES)
                        rows_v[e, sl] = rows_v[e, sl] * wk
                return carry2

            lax.fori_loop(0, _NGRP, mul, 0)
            pltpu.sync_copy(rows_v, acc.at[dst_v.at[ci]], add=True)
            return carry

        lax.fori_loop(0, _NCHUNK, chunk, 0)
        plsc.subcore_barrier()
        for r in range(_RPT // _ZROWS):
            ro = row0 + r * _ZROWS
            pltpu.sync_copy(acc.at[pl.ds(ro, _ZROWS)], out_hbm.at[c, pl.ds(ro, _ZROWS)])

    return k(hs, src2, dst2, w2)


def _tc_layer1(degp, x, w1):
    def body(deg_ref, x_ref, w1_ref, hs_ref, dinv_ref):
        deg = deg_ref[0, :_N] + deg_ref[1, :_N] + 1.0
        dinv = lax.rsqrt(deg)[:, 0:1]
        h = jnp.dot(x_ref[...], w1_ref[...], preferred_element_type=jnp.float32)
        hs_ref[...] = h * dinv
        dinv_ref[...] = dinv

    return pl.pallas_call(
        body,
        out_shape=(
            jax.ShapeDtypeStruct((_N, _HID), jnp.float32),
            jax.ShapeDtypeStruct((_N, 1), jnp.float32),
        ),
    )(degp, x, w1)


def _tc_layer2(agg1, hs, dinv, b1, w2p):
    def body(a_ref, hs_ref, dinv_ref, b1_ref, w2_ref, out_ref):
        t = (a_ref[0, :_N] + a_ref[1, :_N] + hs_ref[...]) * dinv_ref[...] + b1_ref[...]
        r = jnp.maximum(t, 0.0)
        h2 = jnp.dot(r, w2_ref[...], preferred_element_type=jnp.float32)
        out_ref[...] = h2 * dinv_ref[...]

    return pl.pallas_call(
        body,
        out_shape=jax.ShapeDtypeStruct((_N, _CP), jnp.float32),
    )(agg1, hs, dinv, b1, w2p)


def _tc_out(agg2, hs2, dinv, b2):
    def body(a_ref, hs2_ref, dinv_ref, b2_ref, out_ref):
        t = (a_ref[0, :_N] + a_ref[1, :_N] + hs2_ref[...]) * dinv_ref[...]
        o = t[:, :_CLS] + b2_ref[...]
        m = jnp.max(o, axis=1, keepdims=True)
        z = o - m
        lse = jnp.log(jnp.sum(jnp.exp(z), axis=1, keepdims=True))
        out_ref[...] = z - lse

    return pl.pallas_call(
        body,
        out_shape=jax.ShapeDtypeStruct((_N, _CLS), jnp.float32),
    )(agg2, hs2, dinv, b2)


def kernel(x, edge_index, edge_weight, W1, b1, W2, b2):
    src2 = edge_index[0].astype(jnp.int32).reshape(_NW, _NCHUNK, _CHUNK)
    dst2 = edge_index[1].astype(jnp.int32).reshape(_NW, _NCHUNK, _CHUNK)
    w2 = edge_weight.astype(jnp.float32).reshape(_NW, _NCHUNK, _CHUNK)
    x = x.astype(jnp.float32)

    degp = _sc_deg(dst2, w2)
    hs, dinv = _tc_layer1(degp, x, W1)
    agg1 = _sc_agg(hs, src2, dst2, w2, _HID)
    w2p = jnp.pad(W2, ((0, 0), (0, _CP - _CLS)))
    hs2 = _tc_layer2(agg1, hs, dinv, b1.reshape(1, _HID), w2p)
    agg2 = _sc_agg(hs2, src2, dst2, w2, _CP)
    return _tc_out(agg2, hs2, dinv, b2.reshape(1, _CLS))


# speedup vs baseline: 34.3228x; 2.0411x over previous
import functools

import jax
import jax.numpy as jnp
from jax import lax
from jax.experimental import pallas as pl
from jax.experimental.pallas import tpu as pltpu
from jax.experimental.pallas import tpu_sc as plsc

_N = 10000
_E = 320000
_FIN = 128
_HID = 64
_CLS = 40
_CP = 48

_NCORE, _NSUB, _LANES = 2, 16, 16
_NW = _NCORE * _NSUB
_EPT = _E // _NW
_CHUNK = 80
_NCHUNK = _EPT // _CHUNK
_NGRP = _CHUNK // _LANES
_NPAD = 10240
_RPT = _NPAD // _NSUB
_ZROWS = 128

_MESH = dict(core_axis_name="c", subcore_axis_name="s")
_SC_PARAMS = pltpu.CompilerParams(
    use_tc_tiling_on_sc=False, needs_layout_passes=False
)


def _sc_deg(dst2, w2):
    @functools.partial(
        pl.kernel,
        mesh=plsc.VectorSubcoreMesh(**_MESH),
        out_type=jax.ShapeDtypeStruct((_NCORE, _NPAD, _LANES), jnp.float32),
        scratch_types=[
            pltpu.VMEM((_NCHUNK, _CHUNK), jnp.int32),
            pltpu.VMEM((_NCHUNK, _CHUNK), jnp.float32),
            pltpu.VMEM((_CHUNK, _LANES), jnp.float32),
            pltpu.VMEM((_ZROWS, _LANES), jnp.float32),
            pltpu.VMEM_SHARED((_NPAD, _LANES), jnp.float32),
        ],
        compiler_params=_SC_PARAMS,
    )
    def k(dst_hbm, w_hbm, out_hbm, dst_v, w_v, vbuf, zbuf, acc):
        c = lax.axis_index("c")
        s = lax.axis_index("s")
        wid = c * _NSUB + s
        zero = jnp.zeros((_LANES,), jnp.float32)

        def zrow(i, carry):
            zbuf[i, :] = zero
            return carry

        lax.fori_loop(0, _ZROWS, zrow, 0)
        row0 = s * _RPT
        for r in range(_RPT // _ZROWS):
            pltpu.sync_copy(zbuf, acc.at[pl.ds(row0 + r * _ZROWS, _ZROWS)])
        pltpu.sync_copy(dst_hbm.at[wid], dst_v)
        pltpu.sync_copy(w_hbm.at[wid], w_v)
        plsc.subcore_barrier()

        def chunk(ci, carry):
            def fill(g, carry2):
                wv = w_v[ci, pl.ds(g * _LANES, _LANES)]
                for kk in range(_LANES):
                    vbuf[g * _LANES + kk, :] = zero + wv[kk]
                return carry2

            lax.fori_loop(0, _NGRP, fill, 0)
            pltpu.sync_copy(vbuf, acc.at[dst_v.at[ci]], add=True)
            return carry

        lax.fori_loop(0, _NCHUNK, chunk, 0)
        plsc.subcore_barrier()
        for r in range(_RPT // _ZROWS):
            ro = row0 + r * _ZROWS
            pltpu.sync_copy(acc.at[pl.ds(ro, _ZROWS)], out_hbm.at[c, pl.ds(ro, _ZROWS)])

    return k(dst2, w2)


def _sc_agg(hs, src2, dst2, w2, d):
    nj = d // _LANES

    @functools.partial(
        pl.kernel,
        mesh=plsc.VectorSubcoreMesh(**_MESH),
        out_type=jax.ShapeDtypeStruct((_NCORE, _NPAD, d), jnp.float32),
        scratch_types=[
            pltpu.VMEM((_NCHUNK, _CHUNK), jnp.int32),
            pltpu.VMEM((_NCHUNK, _CHUNK), jnp.int32),
            pltpu.VMEM((_NCHUNK, _CHUNK), jnp.float32),
            pltpu.VMEM((_CHUNK, d), jnp.float32),
            pltpu.VMEM((_CHUNK, d), jnp.float32),
            pltpu.VMEM((_CHUNK, d), jnp.float32),
            pltpu.VMEM((_CHUNK, d), jnp.float32),
            pltpu.VMEM((_ZROWS, d), jnp.float32),
            pltpu.VMEM_SHARED((_NPAD, d), jnp.float32),
            pltpu.SemaphoreType.DMA,
            pltpu.SemaphoreType.DMA,
            pltpu.SemaphoreType.DMA,
            pltpu.SemaphoreType.DMA,
        ],
        compiler_params=_SC_PARAMS,
    )
    def k(hs_hbm, src_hbm, dst_hbm, w_hbm, out_hbm,
          src_v, dst_v, w_v, gbuf0, gbuf1, sbuf0, sbuf1, zbuf, acc,
          gsem0, gsem1, ssem0, ssem1):
        c = lax.axis_index("c")
        s = lax.axis_index("s")
        wid = c * _NSUB + s
        zero = jnp.zeros((_LANES,), jnp.float32)
        gbuf = (gbuf0, gbuf1)
        sbuf = (sbuf0, sbuf1)
        gsem = (gsem0, gsem1)
        ssem = (ssem0, ssem1)

        def zrow(i, carry):
            for j in range(nj):
                zbuf[i, pl.ds(j * _LANES, _LANES)] = zero
            return carry

        lax.fori_loop(0, _ZROWS, zrow, 0)
        row0 = s * _RPT
        for r in range(_RPT // _ZROWS):
            pltpu.sync_copy(zbuf, acc.at[pl.ds(row0 + r * _ZROWS, _ZROWS)])
        pltpu.sync_copy(src_hbm.at[wid], src_v)
        pltpu.sync_copy(dst_hbm.at[wid], dst_v)
        pltpu.sync_copy(w_hbm.at[wid], w_v)
        plsc.subcore_barrier()

        def mul_into(ci, gb, sb):
            def mul(g, carry2):
                wv = w_v[ci, pl.ds(g * _LANES, _LANES)]
                for kk in range(_LANES):
                    wk = wv[kk]
                    e = g * _LANES + kk
                    for j in range(nj):
                        sl = pl.ds(j * _LANES, _LANES)
                        sb[e, sl] = gb[e, sl] * wk
                return carry2

            lax.fori_loop(0, _NGRP, mul, 0)

        def gather_start(ci, b):
            pltpu.async_copy(hs_hbm.at[src_v.at[ci]], gbuf[b], gsem[b])

        def gather_wait(ci, b):
            pltpu.make_async_copy(
                hs_hbm.at[src_v.at[ci]], gbuf[b], gsem[b]
            ).wait()

        def scatter_start(ci, b):
            pltpu.async_copy(sbuf[b], acc.at[dst_v.at[ci]], ssem[b], add=True)

        def scatter_wait(ci, b):
            pltpu.make_async_copy(
                sbuf[b], acc.at[dst_v.at[ci]], ssem[b]
            ).wait()

        gather_start(0, 0)
        gather_start(1, 1)

        def pair(i, carry):
            for b in range(2):
                ci = 2 * i + b
                gather_wait(ci, b)

                @pl.when(i > 0)
                def _():
                    scatter_wait(ci - 2, b)

                mul_into(ci, gbuf[b], sbuf[b])

                @pl.when(ci + 2 < _NCHUNK)
                def _():
                    gather_start(ci + 2, b)

                scatter_start(ci, b)
            return carry

        npair = (_NCHUNK - 1) // 2
        lax.fori_loop(0, npair, pair, 0)
        last = _NCHUNK - 1
        gather_wait(last, 0)
        scatter_wait(last - 2, 0)
        mul_into(last, gbuf[0], sbuf[0])
        scatter_start(last, 0)
        scatter_wait(last - 1, 1)
        scatter_wait(last, 0)
        plsc.subcore_barrier()
        for r in range(_RPT // _ZROWS):
            ro = row0 + r * _ZROWS
            pltpu.sync_copy(acc.at[pl.ds(ro, _ZROWS)], out_hbm.at[c, pl.ds(ro, _ZROWS)])

    return k(hs, src2, dst2, w2)


def _tc_layer1(degp, x, w1):
    def body(deg_ref, x_ref, w1_ref, hs_ref, dinv_ref):
        deg = deg_ref[0, :_N] + deg_ref[1, :_N] + 1.0
        dinv = lax.rsqrt(deg)[:, 0:1]
        h = jnp.dot(x_ref[...], w1_ref[...], preferred_element_type=jnp.float32)
        hs_ref[...] = h * dinv
        dinv_ref[...] = dinv

    return pl.pallas_call(
        body,
        out_shape=(
            jax.ShapeDtypeStruct((_N, _HID), jnp.float32),
            jax.ShapeDtypeStruct((_N, 1), jnp.float32),
        ),
    )(degp, x, w1)


def _tc_layer2(agg1, hs, dinv, b1, w2p):
    def body(a_ref, hs_ref, dinv_ref, b1_ref, w2_ref, out_ref):
        t = (a_ref[0, :_N] + a_ref[1, :_N] + hs_ref[...]) * dinv_ref[...] + b1_ref[...]
        r = jnp.maximum(t, 0.0)
        h2 = jnp.dot(r, w2_ref[...], preferred_element_type=jnp.float32)
        out_ref[...] = h2 * dinv_ref[...]

    return pl.pallas_call(
        body,
        out_shape=jax.ShapeDtypeStruct((_N, _CP), jnp.float32),
    )(agg1, hs, dinv, b1, w2p)


def _tc_out(agg2, hs2, dinv, b2):
    def body(a_ref, hs2_ref, dinv_ref, b2_ref, out_ref):
        t = (a_ref[0, :_N] + a_ref[1, :_N] + hs2_ref[...]) * dinv_ref[...]
        o = t[:, :_CLS] + b2_ref[...]
        m = jnp.max(o, axis=1, keepdims=True)
        z = o - m
        lse = jnp.log(jnp.sum(jnp.exp(z), axis=1, keepdims=True))
        out_ref[...] = z - lse

    return pl.pallas_call(
        body,
        out_shape=jax.ShapeDtypeStruct((_N, _CLS), jnp.float32),
    )(agg2, hs2, dinv, b2)


def kernel(x, edge_index, edge_weight, W1, b1, W2, b2):
    src2 = edge_index[0].astype(jnp.int32).reshape(_NW, _NCHUNK, _CHUNK)
    dst2 = edge_index[1].astype(jnp.int32).reshape(_NW, _NCHUNK, _CHUNK)
    w2 = edge_weight.astype(jnp.float32).reshape(_NW, _NCHUNK, _CHUNK)
    x = x.astype(jnp.float32)

    degp = _sc_deg(dst2, w2)
    hs, dinv = _tc_layer1(degp, x, W1)
    agg1 = _sc_agg(hs, src2, dst2, w2, _HID)
    w2p = jnp.pad(W2, ((0, 0), (0, _CP - _CLS)))
    hs2 = _tc_layer2(agg1, hs, dinv, b1.reshape(1, _HID), w2p)
    agg2 = _sc_agg(hs2, src2, dst2, w2, _CP)
    return _tc_out(agg2, hs2, dinv, b2.reshape(1, _CLS))
